# baseline (speedup 1.0000x reference)
import jax
import jax.numpy as jnp
from jax.experimental import pallas as pl
from jax.experimental.pallas import tpu as pltpu

_N = 300
_B = 32
_DIN = 64
_DH = 128
_DOUT = 32
_GC = 40
_ALPHA = 3.0
_GRP = 8


def _body(emb1_ref, emb2_ref, l1w_ref, l1b_ref, l2w_ref, l2b_ref,
          g1w_ref, g1b_ref, g2w_ref, g2b_ref, x_ref, out_ref, t_ref, g_ref,
          xw_ref, hw_ref):
    b = pl.program_id(0)

    @pl.when(b == 0)
    def _build_t():
        gu = jax.lax.broadcasted_iota(jnp.int32, (_GRP * _DOUT, _GRP * _DOUT), 0)
        gv = jax.lax.broadcasted_iota(jnp.int32, (_GRP * _DOUT, _GRP * _DOUT), 1)
        g_ref[...] = ((gu // _DOUT) == (gv // _DOUT)).astype(jnp.float32)
        n1 = jnp.tanh(_ALPHA * (
            jax.lax.dot_general(emb1_ref[...], l1w_ref[...],
                                (((1,), (0,)), ((), ())),
                                preferred_element_type=jnp.float32)
            + l1b_ref[...]))
        n2 = jnp.tanh(_ALPHA * (
            jax.lax.dot_general(emb2_ref[...], l2w_ref[...],
                                (((1,), (0,)), ((), ())),
                                preferred_element_type=jnp.float32)
            + l2b_ref[...]))
        p = jax.lax.dot_general(n1, n2, (((1,), (1,)), ((), ())),
                                preferred_element_type=jnp.float32)
        q = jax.lax.dot_general(n2, n1, (((1,), (1,)), ((), ())),
                                preferred_element_type=jnp.float32)
        eye = (jax.lax.broadcasted_iota(jnp.int32, (_N, _N), 0)
               == jax.lax.broadcasted_iota(jnp.int32, (_N, _N), 1))
        eyef = eye.astype(jnp.float32)
        ah = (p > q).astype(jnp.float32) + eyef
        aht = (q > p).astype(jnp.float32) + eyef
        ones_c = jnp.ones((_N, 1), dtype=jnp.float32)
        ones_r = jnp.ones((1, _N), dtype=jnp.float32)
        deg_c = jax.lax.dot_general(aht, ones_c, (((1,), (0,)), ((), ())),
                                    preferred_element_type=jnp.float32)
        deg_r = jax.lax.dot_general(ones_r, ah, (((1,), (0,)), ((), ())),
                                    preferred_element_type=jnp.float32)
        dinv_c = 1.0 / jnp.sqrt(deg_c)
        dinv_r = 1.0 / jnp.sqrt(deg_r)
        t_ref[...] = aht * dinv_c * dinv_r

    t = t_ref[...]
    for i in range(_GRP):
        xw_ref[:, i * _DH:(i + 1) * _DH] = jax.lax.dot_general(
            x_ref[i], g1w_ref[...], (((1,), (0,)), ((), ())),
            preferred_element_type=jnp.float32)
    h = jax.lax.dot_general(t, xw_ref[...], (((1,), (0,)), ((), ())),
                            preferred_element_type=jnp.float32)
    for i in range(_GRP):
        hw_ref[:, i * _DOUT:(i + 1) * _DOUT] = jax.lax.dot_general(
            jnp.maximum(h[:, i * _DH:(i + 1) * _DH] + g1b_ref[...], 0.0),
            g2w_ref[...], (((1,), (0,)), ((), ())),
            preferred_element_type=jnp.float32)
    o = jax.lax.dot_general(t, hw_ref[...], (((1,), (0,)), ((), ())),
                            preferred_element_type=jnp.float32)
    o = o + jnp.tile(g2b_ref[...], (1, _GRP))
    e = jnp.exp(o - jnp.max(o, axis=1, keepdims=True))
    s = jax.lax.dot_general(e, g_ref[...], (((1,), (0,)), ((), ())),
                            preferred_element_type=jnp.float32)
    r = e / s
    for i in range(_GRP):
        out_ref[i] = r[:, i * _DOUT:(i + 1) * _DOUT]


def kernel(x, emb1, emb2, lin1_W, lin1_b, lin2_W, lin2_b,
           gcn1_W, gcn1_b, gcn2_W, gcn2_b):
    x = x.astype(jnp.float32).reshape(_B, _N, _DIN)
    l1b = lin1_b.reshape(1, _GC)
    l2b = lin2_b.reshape(1, _GC)
    g1b = gcn1_b.reshape(1, _DH)
    g2b = gcn2_b.reshape(1, _DOUT)

    fixed = lambda shape: pl.BlockSpec(shape, lambda b: (0,) * len(shape))
    out = pl.pallas_call(
        _body,
        grid=(_B // _GRP,),
        in_specs=[
            fixed((_N, _GC)), fixed((_N, _GC)),
            fixed((_GC, _GC)), fixed((1, _GC)),
            fixed((_GC, _GC)), fixed((1, _GC)),
            fixed((_DIN, _DH)), fixed((1, _DH)),
            fixed((_DH, _DOUT)), fixed((1, _DOUT)),
            pl.BlockSpec((_GRP, _N, _DIN), lambda b: (b, 0, 0)),
        ],
        out_specs=pl.BlockSpec((_GRP, _N, _DOUT), lambda b: (b, 0, 0)),
        out_shape=jax.ShapeDtypeStruct((_B, _N, _DOUT), jnp.float32),
        scratch_shapes=[pltpu.VMEM((_N, _N), jnp.float32),
                        pltpu.VMEM((_GRP * _DOUT, _GRP * _DOUT), jnp.float32),
                        pltpu.VMEM((_N, _GRP * _DH), jnp.float32),
                        pltpu.VMEM((_N, _GRP * _DOUT), jnp.float32)],
    )(emb1, emb2, lin1_W, l1b, lin2_W, l2b, gcn1_W, g1b, gcn2_W, g2b, x)
    return out.reshape(_B * _N, _DOUT)


# speedup vs baseline: 1.0181x; 1.0181x over previous
import jax
import jax.numpy as jnp
from jax.experimental import pallas as pl
from jax.experimental.pallas import tpu as pltpu

_N = 300
_B = 32
_DIN = 64
_DH = 128
_DOUT = 32
_GC = 40
_ALPHA = 3.0
_GRP = 16


def _body(emb1_ref, emb2_ref, l1w_ref, l1b_ref, l2w_ref, l2b_ref,
          g1w_ref, g1b_ref, g2w_ref, g2b_ref, x_ref, out_ref, t_ref, g_ref,
          xw_ref, hw_ref):
    b = pl.program_id(0)

    @pl.when(b == 0)
    def _build_t():
        gu = jax.lax.broadcasted_iota(jnp.int32, (_GRP * _DOUT, _GRP * _DOUT), 0)
        gv = jax.lax.broadcasted_iota(jnp.int32, (_GRP * _DOUT, _GRP * _DOUT), 1)
        g_ref[...] = ((gu // _DOUT) == (gv // _DOUT)).astype(jnp.float32)
        n1 = jnp.tanh(_ALPHA * (
            jax.lax.dot_general(emb1_ref[...], l1w_ref[...],
                                (((1,), (0,)), ((), ())),
                                preferred_element_type=jnp.float32)
            + l1b_ref[...]))
        n2 = jnp.tanh(_ALPHA * (
            jax.lax.dot_general(emb2_ref[...], l2w_ref[...],
                                (((1,), (0,)), ((), ())),
                                preferred_element_type=jnp.float32)
            + l2b_ref[...]))
        p = jax.lax.dot_general(n1, n2, (((1,), (1,)), ((), ())),
                                preferred_element_type=jnp.float32)
        q = jax.lax.dot_general(n2, n1, (((1,), (1,)), ((), ())),
                                preferred_element_type=jnp.float32)
        eye = (jax.lax.broadcasted_iota(jnp.int32, (_N, _N), 0)
               == jax.lax.broadcasted_iota(jnp.int32, (_N, _N), 1))
        eyef = eye.astype(jnp.float32)
        ah = (p > q).astype(jnp.float32) + eyef
        aht = (q > p).astype(jnp.float32) + eyef
        ones_c = jnp.ones((_N, 1), dtype=jnp.float32)
        ones_r = jnp.ones((1, _N), dtype=jnp.float32)
        deg_c = jax.lax.dot_general(aht, ones_c, (((1,), (0,)), ((), ())),
                                    preferred_element_type=jnp.float32)
        deg_r = jax.lax.dot_general(ones_r, ah, (((1,), (0,)), ((), ())),
                                    preferred_element_type=jnp.float32)
        dinv_c = 1.0 / jnp.sqrt(deg_c)
        dinv_r = 1.0 / jnp.sqrt(deg_r)
        t_ref[...] = aht * dinv_c * dinv_r

    t = t_ref[...]
    for i in range(_GRP):
        xw_ref[:, i * _DH:(i + 1) * _DH] = jax.lax.dot_general(
            x_ref[i], g1w_ref[...], (((1,), (0,)), ((), ())),
            preferred_element_type=jnp.float32)
    h = jax.lax.dot_general(t, xw_ref[...], (((1,), (0,)), ((), ())),
                            preferred_element_type=jnp.float32)
    for i in range(_GRP):
        hw_ref[:, i * _DOUT:(i + 1) * _DOUT] = jax.lax.dot_general(
            jnp.maximum(h[:, i * _DH:(i + 1) * _DH] + g1b_ref[...], 0.0),
            g2w_ref[...], (((1,), (0,)), ((), ())),
            preferred_element_type=jnp.float32)
    o = jax.lax.dot_general(t, hw_ref[...], (((1,), (0,)), ((), ())),
                            preferred_element_type=jnp.float32)
    o = o + jnp.tile(g2b_ref[...], (1, _GRP))
    e = jnp.exp(o - jnp.max(o, axis=1, keepdims=True))
    s = jax.lax.dot_general(e, g_ref[...], (((1,), (0,)), ((), ())),
                            preferred_element_type=jnp.float32)
    r = e / s
    for i in range(_GRP):
        out_ref[i] = r[:, i * _DOUT:(i + 1) * _DOUT]


def kernel(x, emb1, emb2, lin1_W, lin1_b, lin2_W, lin2_b,
           gcn1_W, gcn1_b, gcn2_W, gcn2_b):
    x = x.astype(jnp.float32).reshape(_B, _N, _DIN)
    l1b = lin1_b.reshape(1, _GC)
    l2b = lin2_b.reshape(1, _GC)
    g1b = gcn1_b.reshape(1, _DH)
    g2b = gcn2_b.reshape(1, _DOUT)

    fixed = lambda shape: pl.BlockSpec(shape, lambda b: (0,) * len(shape))
    out = pl.pallas_call(
        _body,
        grid=(_B // _GRP,),
        in_specs=[
            fixed((_N, _GC)), fixed((_N, _GC)),
            fixed((_GC, _GC)), fixed((1, _GC)),
            fixed((_GC, _GC)), fixed((1, _GC)),
            fixed((_DIN, _DH)), fixed((1, _DH)),
            fixed((_DH, _DOUT)), fixed((1, _DOUT)),
            pl.BlockSpec((_GRP, _N, _DIN), lambda b: (b, 0, 0)),
        ],
        out_specs=pl.BlockSpec((_GRP, _N, _DOUT), lambda b: (b, 0, 0)),
        out_shape=jax.ShapeDtypeStruct((_B, _N, _DOUT), jnp.float32),
        scratch_shapes=[pltpu.VMEM((_N, _N), jnp.float32),
                        pltpu.VMEM((_GRP * _DOUT, _GRP * _DOUT), jnp.float32),
                        pltpu.VMEM((_N, _GRP * _DH), jnp.float32),
                        pltpu.VMEM((_N, _GRP * _DOUT), jnp.float32)],
    )(emb1, emb2, lin1_W, l1b, lin2_W, l2b, gcn1_W, g1b, gcn2_W, g2b, x)
    return out.reshape(_B * _N, _DOUT)
